# baseline (speedup 1.0000x reference)
import functools

import jax
import jax.numpy as jnp
from jax import lax
from jax.experimental import pallas as pl
from jax.experimental.pallas import tpu as pltpu
from jax.experimental.pallas import tpu_sc as plsc

E = 8
H = 1024
HI = H // 2
F = 512
T = 2048
A = 2 * T
TM = 512
GMAX = A // TM + E
P = GMAX * TM
CH = 512
NW = 32
TPW = T // NW
YSH = 1032
RP2 = P // 16
CC = 64

_SC_PARAMS = pltpu.CompilerParams(needs_layout_passes=False)


def _gate_kernel(x_ref, gw_ref, slot_ref, w_ref, meta_ref, xbi_ref):
    x = x_ref[...]
    gw = gw_ref[...]
    logits = lax.dot_general(
        x, gw, (((1,), (1,)), ((), ())), preferred_element_type=jnp.float32)
    m = jnp.max(logits, axis=1, keepdims=True)
    p = jnp.exp(logits - m)
    p = p / jnp.sum(p, axis=1, keepdims=True)

    lane = lax.broadcasted_iota(jnp.int32, p.shape, 1)
    m1 = jnp.max(p, axis=1, keepdims=True)
    i1 = jnp.min(jnp.where(p >= m1, lane, E), axis=1, keepdims=True)
    p2 = jnp.where(lane == i1, -jnp.inf, p)
    m2 = jnp.max(p2, axis=1, keepdims=True)
    i2 = jnp.min(jnp.where(p2 >= m2, lane, E), axis=1, keepdims=True)
    s = m1 + m2 + 1e-20
    w1, w2 = m1 / s, m2 / s

    o1 = (lane == i1).astype(jnp.float32)
    o2 = (lane == i2).astype(jnp.float32)
    onehot = jnp.concatenate([o1, o2], axis=0)

    r = lax.broadcasted_iota(jnp.int32, (CH, CH), 0)
    c = lax.broadcasted_iota(jnp.int32, (CH, CH), 1)
    lt = (c < r).astype(jnp.bfloat16)
    carry = jnp.zeros((1, E), jnp.float32)
    ranks = []
    for i in range(A // CH):
        blk = onehot[i * CH:(i + 1) * CH, :]
        within = lax.dot_general(
            lt, blk.astype(jnp.bfloat16), (((1,), (0,)), ((), ())),
            preferred_element_type=jnp.float32)
        ranks.append(within + carry)
        carry = carry + jnp.sum(blk, axis=0, keepdims=True)
    ranks = jnp.concatenate(ranks, axis=0)
    cnt = carry

    pt = jnp.floor((cnt + (TM - 1)) / TM)
    er = lax.broadcasted_iota(jnp.int32, (E, E), 0)
    ec = lax.broadcasted_iota(jnp.int32, (E, E), 1)
    upper = (er < ec).astype(jnp.float32)
    ts_row = lax.dot_general(
        pt, upper, (((1,), (0,)), ((), ())), preferred_element_type=jnp.float32)
    po_row = ts_row * TM

    rank_a = jnp.sum(ranks * onehot, axis=1, keepdims=True)
    off_a = jnp.sum(po_row * onehot, axis=1, keepdims=True)
    slot_ref[...] = (rank_a + off_a).astype(jnp.int32)
    w_ref[...] = jnp.concatenate([w1, w2], axis=0)

    ptb = jnp.broadcast_to(pt, (E, E))
    ca_col = jnp.sum(jnp.where(ec <= er, ptb, 0.0), axis=1, keepdims=True)
    g_lane = lax.broadcasted_iota(jnp.int32, (E, 128), 1)
    te = jnp.sum((g_lane >= ca_col.astype(jnp.int32)).astype(jnp.float32),
                 axis=0, keepdims=True)
    te = jnp.minimum(te, float(E - 1))
    g_used = jnp.sum(pt)
    lane128 = lax.broadcasted_iota(jnp.int32, (1, 128), 1)
    meta_ref[...] = jnp.where(lane128 == 127, g_used, te).astype(jnp.int32)

    xb = x.astype(jnp.bfloat16).reshape(2 * T, HI)
    xbi_ref[...] = pltpu.bitcast(xb, jnp.int32)


def _dispatch_kernel(x_hbm, slot_hbm, w_hbm, xs_hbm, ts_hbm, ws_hbm,
                     slot_v, w_v, ts_v, ws_v, idx_v, buf, sem, semi, semb):
    cid = lax.axis_index("c")
    sid = lax.axis_index("s")
    wid = sid * 2 + cid
    t0 = wid * TPW
    pltpu.async_copy(slot_hbm.at[pl.ds(t0, TPW)], idx_v.at[0], semi)
    pltpu.async_copy(slot_hbm.at[pl.ds(T + t0, TPW)], idx_v.at[1], semi)
    cp = pltpu.async_copy(x_hbm.at[pl.ds(t0, TPW)], buf, semb)
    pltpu.make_async_copy(slot_hbm.at[pl.ds(0, TPW)], idx_v.at[0], semi).wait()
    pltpu.make_async_copy(slot_hbm.at[pl.ds(0, TPW)], idx_v.at[1], semi).wait()
    cp.wait()
    pltpu.sync_copy(buf, xs_hbm.at[idx_v.at[0]])
    pltpu.sync_copy(buf, xs_hbm.at[idx_v.at[1]])

    @pl.when(jnp.logical_and(cid == 0, sid == 0))
    def _():
        pltpu.async_copy(slot_hbm, slot_v, sem).wait()

        @pl.loop(0, P, step=16)
        def _(i):
            ts_v[pl.ds(i, 16)] = (lax.iota(jnp.int32, 16) + i) % 2048 + A

        @pl.loop(0, A, step=16)
        def _(a):
            sl = slot_v[pl.ds(a, 16)]
            av = lax.iota(jnp.int32, 16) + a
            plsc.store_scatter(ts_v, [sl], av)

        pltpu.async_copy(ts_v, ts_hbm, sem).wait()

    @pl.when(jnp.logical_and(cid == 1, sid == 0))
    def _():
        pltpu.async_copy(slot_hbm, slot_v, sem).wait()
        pltpu.async_copy(w_hbm, w_v, sem).wait()

        @pl.loop(0, P, step=16)
        def _(i):
            ws_v[pl.ds(i, 16)] = jnp.zeros((16,), jnp.float32)

        @pl.loop(0, A, step=16)
        def _(a):
            sl = slot_v[pl.ds(a, 16)]
            wv = w_v[pl.ds(a, 16)]
            plsc.store_scatter(ws_v, [sl], wv)

        pltpu.async_copy(ws_v, ws_hbm, sem).wait()


def _moe_kernel(meta_ref, xs_ref, ws_ref, wg_ref, wu_ref, wd_ref, ys_ref):
    g = pl.program_id(0)
    used = g < meta_ref[0, 127]

    @pl.when(used)
    def _():
        xb = pltpu.bitcast(xs_ref[...], jnp.bfloat16).reshape(TM, H)
        wg = wg_ref[0].astype(jnp.bfloat16)
        wu = wu_ref[0].astype(jnp.bfloat16)
        wd = wd_ref[0].astype(jnp.bfloat16)
        wfull = ws_ref[0, 0, :][:, None]
        SB = TM // 2
        for q in range(2):
            xq = xb[q * SB:(q + 1) * SB]
            gg = lax.dot_general(
                xq, wg, (((1,), (1,)), ((), ())),
                preferred_element_type=jnp.float32)
            uu = lax.dot_general(
                xq, wu, (((1,), (1,)), ((), ())),
                preferred_element_type=jnp.float32)
            a = (gg * jax.nn.sigmoid(gg) * uu).astype(jnp.bfloat16)
            o = lax.dot_general(
                a, wd, (((1,), (1,)), ((), ())),
                preferred_element_type=jnp.float32)
            w = wfull[q * SB:(q + 1) * SB]
            yb = jnp.where(w > 0, o * w, 0.0).astype(jnp.bfloat16)
            ys_ref[pl.ds(q * SB, SB), :] = pltpu.bitcast(
                yb.reshape(2 * SB, HI), jnp.int32)


def _combine_scatter_kernel(ys_hbm, ds_hbm, r_hbm,
                            idx_v, buf0, buf1, semi, sem0, sem1):
    cid = lax.axis_index("c")
    sid = lax.axis_index("s")
    wid = sid * 2 + cid
    rpt = P // NW
    nck = rpt // CC
    base = wid * rpt
    for ci in range(nck):
        pltpu.async_copy(ds_hbm.at[pl.ds(base + ci * CC, CC)],
                         idx_v.at[ci], semi).wait()

    bufs = (buf0, buf1)
    sems = (sem0, sem1)
    cps = [None, None]
    cps[0] = pltpu.async_copy(ys_hbm.at[pl.ds(base, CC)], buf0, sem0)
    for ci in range(nck):
        if ci + 1 < nck:
            nb = (ci + 1) % 2
            cps[nb] = pltpu.async_copy(
                ys_hbm.at[pl.ds(base + (ci + 1) * CC, CC)], bufs[nb], sems[nb])
        cps[ci % 2].wait()
        pltpu.sync_copy(bufs[ci % 2], r_hbm.at[idx_v.at[ci]])


def _add_kernel(r0_ref, r1_ref, y_ref):
    r0 = pltpu.bitcast(r0_ref[0], jnp.bfloat16).reshape(TM, H)
    r1 = pltpu.bitcast(r1_ref[0], jnp.bfloat16).reshape(TM, H)
    y_ref[...] = r0.astype(jnp.float32) + r1.astype(jnp.float32)


def kernel(hidden_states, gate_w, Wg, Wu, Wd):
    orig_shape = hidden_states.shape
    x = hidden_states.reshape(-1, H)

    slot, w_flat, meta, xbi = pl.pallas_call(
        _gate_kernel,
        out_shape=(jax.ShapeDtypeStruct((A, 1), jnp.int32),
                   jax.ShapeDtypeStruct((A, 1), jnp.float32),
                   jax.ShapeDtypeStruct((1, 128), jnp.int32),
                   jax.ShapeDtypeStruct((T, HI), jnp.int32)),
        in_specs=[pl.BlockSpec((T, H), lambda: (0, 0)),
                  pl.BlockSpec((E, H), lambda: (0, 0))],
        out_specs=(pl.BlockSpec((A, 1), lambda: (0, 0)),
                   pl.BlockSpec((A, 1), lambda: (0, 0)),
                   pl.BlockSpec((1, 128), lambda: (0, 0)),
                   pl.BlockSpec((T, HI), lambda: (0, 0))),
    )(x, gate_w)

    slot1 = slot.reshape(A)
    wf1 = w_flat.reshape(A)

    xs_i, dst, w_slot = pl.kernel(
        _dispatch_kernel,
        out_type=(jax.ShapeDtypeStruct((P, HI), jnp.int32),
                  jax.ShapeDtypeStruct((P,), jnp.int32),
                  jax.ShapeDtypeStruct((P,), jnp.float32)),
        mesh=plsc.VectorSubcoreMesh(core_axis_name="c", subcore_axis_name="s"),
        compiler_params=_SC_PARAMS,
        scratch_types=[pltpu.VMEM((A,), jnp.int32),
                       pltpu.VMEM((A,), jnp.float32),
                       pltpu.VMEM((P,), jnp.int32),
                       pltpu.VMEM((P,), jnp.float32),
                       pltpu.VMEM((2, TPW), jnp.int32),
                       pltpu.VMEM((TPW, HI), jnp.int32),
                       pltpu.SemaphoreType.DMA,
                       pltpu.SemaphoreType.DMA,
                       pltpu.SemaphoreType.DMA],
    )(xbi, slot1, wf1)

    ws3 = w_slot.reshape(GMAX, 1, TM)

    ys = pl.pallas_call(
        _moe_kernel,
        grid_spec=pltpu.PrefetchScalarGridSpec(
            num_scalar_prefetch=1,
            grid=(GMAX,),
            in_specs=[
                pl.BlockSpec((TM, HI),
                             lambda g, m: (jnp.minimum(g, m[0, 127] - 1), 0)),
                pl.BlockSpec((1, 1, TM),
                             lambda g, m: (jnp.minimum(g, m[0, 127] - 1), 0, 0)),
                pl.BlockSpec((1, F, H), lambda g, m: (m[0, g], 0, 0)),
                pl.BlockSpec((1, F, H), lambda g, m: (m[0, g], 0, 0)),
                pl.BlockSpec((1, H, F), lambda g, m: (m[0, g], 0, 0)),
            ],
            out_specs=pl.BlockSpec(
                (TM, HI), lambda g, m: (jnp.minimum(g, m[0, 127] - 1), 0)),
        ),
        out_shape=jax.ShapeDtypeStruct((P, HI), jnp.int32),
        compiler_params=pltpu.CompilerParams(
            dimension_semantics=("arbitrary",)),
    )(meta, xs_i, ws3, Wg, Wu, Wd)

    r = pl.kernel(
        _combine_scatter_kernel,
        out_type=jax.ShapeDtypeStruct((P, HI), jnp.int32),
        mesh=plsc.VectorSubcoreMesh(core_axis_name="c", subcore_axis_name="s"),
        compiler_params=_SC_PARAMS,
        scratch_types=[pltpu.VMEM((P // NW // CC, CC), jnp.int32),
                       pltpu.VMEM((CC, HI), jnp.int32),
                       pltpu.VMEM((CC, HI), jnp.int32),
                       pltpu.SemaphoreType.DMA,
                       pltpu.SemaphoreType.DMA,
                       pltpu.SemaphoreType.DMA],
    )(ys, dst)
    r3 = r.reshape(P // T, T, HI)

    y = pl.pallas_call(
        _add_kernel,
        grid=(T // TM,),
        out_shape=jax.ShapeDtypeStruct((T, H), jnp.float32),
        in_specs=[pl.BlockSpec((1, TM, HI), lambda t: (0, t, 0)),
                  pl.BlockSpec((1, TM, HI), lambda t: (1, t, 0))],
        out_specs=pl.BlockSpec((TM, H), lambda t: (t, 0)),
    )(r3, r3)

    return y.reshape(orig_shape)


# speedup vs baseline: 1.0098x; 1.0098x over previous
import functools

import jax
import jax.numpy as jnp
from jax import lax
from jax.experimental import pallas as pl
from jax.experimental.pallas import tpu as pltpu
from jax.experimental.pallas import tpu_sc as plsc

E = 8
H = 1024
HI = H // 2
F = 512
T = 2048
A = 2 * T
TM = 512
GMAX = A // TM + E
P = GMAX * TM
CH = 512
NW = 32
TPW = T // NW
YSH = 1032
RP2 = P // 16
CC = 64

_SC_PARAMS = pltpu.CompilerParams(needs_layout_passes=False)


def _gate_kernel(x_ref, gw_ref, slot_ref, w_ref, meta_ref, xbi_ref):
    x = x_ref[...]
    gw = gw_ref[...]
    logits = lax.dot_general(
        x, gw, (((1,), (1,)), ((), ())), preferred_element_type=jnp.float32)
    m = jnp.max(logits, axis=1, keepdims=True)
    p = jnp.exp(logits - m)
    p = p / jnp.sum(p, axis=1, keepdims=True)

    lane = lax.broadcasted_iota(jnp.int32, p.shape, 1)
    m1 = jnp.max(p, axis=1, keepdims=True)
    i1 = jnp.min(jnp.where(p >= m1, lane, E), axis=1, keepdims=True)
    p2 = jnp.where(lane == i1, -jnp.inf, p)
    m2 = jnp.max(p2, axis=1, keepdims=True)
    i2 = jnp.min(jnp.where(p2 >= m2, lane, E), axis=1, keepdims=True)
    s = m1 + m2 + 1e-20
    w1, w2 = m1 / s, m2 / s

    o1 = (lane == i1).astype(jnp.float32)
    o2 = (lane == i2).astype(jnp.float32)
    onehot = jnp.concatenate([o1, o2], axis=0)

    r = lax.broadcasted_iota(jnp.int32, (CH, CH), 0)
    c = lax.broadcasted_iota(jnp.int32, (CH, CH), 1)
    lt = (c < r).astype(jnp.bfloat16)
    carry = jnp.zeros((1, E), jnp.float32)
    ranks = []
    for i in range(A // CH):
        blk = onehot[i * CH:(i + 1) * CH, :]
        within = lax.dot_general(
            lt, blk.astype(jnp.bfloat16), (((1,), (0,)), ((), ())),
            preferred_element_type=jnp.float32)
        ranks.append(within + carry)
        carry = carry + jnp.sum(blk, axis=0, keepdims=True)
    ranks = jnp.concatenate(ranks, axis=0)
    cnt = carry

    pt = jnp.floor((cnt + (TM - 1)) / TM)
    er = lax.broadcasted_iota(jnp.int32, (E, E), 0)
    ec = lax.broadcasted_iota(jnp.int32, (E, E), 1)
    upper = (er < ec).astype(jnp.float32)
    ts_row = lax.dot_general(
        pt, upper, (((1,), (0,)), ((), ())), preferred_element_type=jnp.float32)
    po_row = ts_row * TM

    rank_a = jnp.sum(ranks * onehot, axis=1, keepdims=True)
    off_a = jnp.sum(po_row * onehot, axis=1, keepdims=True)
    slot_ref[...] = (rank_a + off_a).astype(jnp.int32)
    w_ref[...] = jnp.concatenate([w1, w2], axis=0)

    ptb = jnp.broadcast_to(pt, (E, E))
    ca_col = jnp.sum(jnp.where(ec <= er, ptb, 0.0), axis=1, keepdims=True)
    g_lane = lax.broadcasted_iota(jnp.int32, (E, 128), 1)
    te = jnp.sum((g_lane >= ca_col.astype(jnp.int32)).astype(jnp.float32),
                 axis=0, keepdims=True)
    te = jnp.minimum(te, float(E - 1))
    g_used = jnp.sum(pt)
    lane128 = lax.broadcasted_iota(jnp.int32, (1, 128), 1)
    meta_ref[...] = jnp.where(lane128 == 127, g_used, te).astype(jnp.int32)

    xb = x.astype(jnp.bfloat16).reshape(2 * T, HI)
    xbi_ref[...] = pltpu.bitcast(xb, jnp.int32)


def _dispatch_kernel(x_hbm, slot_hbm, w_hbm, xs_hbm, ts_hbm, ws_hbm,
                     slot_v, w_v, ts_v, ws_v, idx_v, buf, sem, semi, semb):
    cid = lax.axis_index("c")
    sid = lax.axis_index("s")
    wid = sid * 2 + cid
    t0 = wid * TPW
    pltpu.async_copy(slot_hbm.at[pl.ds(t0, TPW)], idx_v.at[0], semi)
    pltpu.async_copy(slot_hbm.at[pl.ds(T + t0, TPW)], idx_v.at[1], semi)
    cp = pltpu.async_copy(x_hbm.at[pl.ds(t0, TPW)], buf, semb)
    pltpu.make_async_copy(slot_hbm.at[pl.ds(0, TPW)], idx_v.at[0], semi).wait()
    pltpu.make_async_copy(slot_hbm.at[pl.ds(0, TPW)], idx_v.at[1], semi).wait()
    cp.wait()
    pltpu.sync_copy(buf, xs_hbm.at[idx_v.at[0]])
    pltpu.sync_copy(buf, xs_hbm.at[idx_v.at[1]])

    @pl.when(jnp.logical_and(cid == 0, sid == 0))
    def _():
        pltpu.async_copy(slot_hbm, slot_v, sem).wait()

        @pl.loop(0, P, step=16)
        def _(i):
            ts_v[pl.ds(i, 16)] = (lax.iota(jnp.int32, 16) + i) % 2048 + A

        @pl.loop(0, A, step=16)
        def _(a):
            sl = slot_v[pl.ds(a, 16)]
            av = lax.iota(jnp.int32, 16) + a
            plsc.store_scatter(ts_v, [sl], av)

        pltpu.async_copy(ts_v, ts_hbm, sem).wait()

    @pl.when(jnp.logical_and(cid == 1, sid == 0))
    def _():
        pltpu.async_copy(slot_hbm, slot_v, sem).wait()
        pltpu.async_copy(w_hbm, w_v, sem).wait()

        @pl.loop(0, P, step=16)
        def _(i):
            ws_v[pl.ds(i, 16)] = jnp.zeros((16,), jnp.float32)

        @pl.loop(0, A, step=16)
        def _(a):
            sl = slot_v[pl.ds(a, 16)]
            wv = w_v[pl.ds(a, 16)]
            plsc.store_scatter(ws_v, [sl], wv)

        pltpu.async_copy(ws_v, ws_hbm, sem).wait()


def _moe_kernel(meta_ref, xs_ref, ws_ref, wg_ref, wu_ref, wd_ref, ys_ref):
    g = pl.program_id(0)
    used = g < meta_ref[0, 127]

    @pl.when(used)
    def _():
        xb = pltpu.bitcast(xs_ref[...], jnp.bfloat16).reshape(TM, H)
        gg = lax.dot_general(
            xb, wg_ref[0].astype(jnp.bfloat16), (((1,), (1,)), ((), ())),
            preferred_element_type=jnp.float32)
        uu = lax.dot_general(
            xb, wu_ref[0].astype(jnp.bfloat16), (((1,), (1,)), ((), ())),
            preferred_element_type=jnp.float32)
        a = (gg * jax.nn.sigmoid(gg) * uu).astype(jnp.bfloat16)
        o = lax.dot_general(
            a, wd_ref[0].astype(jnp.bfloat16), (((1,), (1,)), ((), ())),
            preferred_element_type=jnp.float32)
        w = ws_ref[0, 0, :][:, None]
        yb = jnp.where(w > 0, o * w, 0.0).astype(jnp.bfloat16)
        ys_ref[...] = pltpu.bitcast(yb.reshape(2 * TM, HI), jnp.int32)


def _combine_scatter_kernel(ys_hbm, ds_hbm, r_hbm,
                            idx_v, buf0, buf1, semi, sem0, sem1):
    cid = lax.axis_index("c")
    sid = lax.axis_index("s")
    wid = sid * 2 + cid
    rpt = P // NW
    nck = rpt // CC
    base = wid * rpt
    for ci in range(nck):
        pltpu.async_copy(ds_hbm.at[pl.ds(base + ci * CC, CC)],
                         idx_v.at[ci], semi).wait()

    bufs = (buf0, buf1)
    sems = (sem0, sem1)
    cps = [None, None]
    cps[0] = pltpu.async_copy(ys_hbm.at[pl.ds(base, CC)], buf0, sem0)
    for ci in range(nck):
        if ci + 1 < nck:
            nb = (ci + 1) % 2
            cps[nb] = pltpu.async_copy(
                ys_hbm.at[pl.ds(base + (ci + 1) * CC, CC)], bufs[nb], sems[nb])
        cps[ci % 2].wait()
        pltpu.sync_copy(bufs[ci % 2], r_hbm.at[idx_v.at[ci]])


def _add_kernel(r0_ref, r1_ref, y_ref):
    r0 = pltpu.bitcast(r0_ref[0], jnp.bfloat16).reshape(TM, H)
    r1 = pltpu.bitcast(r1_ref[0], jnp.bfloat16).reshape(TM, H)
    y_ref[...] = r0.astype(jnp.float32) + r1.astype(jnp.float32)


def kernel(hidden_states, gate_w, Wg, Wu, Wd):
    orig_shape = hidden_states.shape
    x = hidden_states.reshape(-1, H)

    slot, w_flat, meta, xbi = pl.pallas_call(
        _gate_kernel,
        out_shape=(jax.ShapeDtypeStruct((A, 1), jnp.int32),
                   jax.ShapeDtypeStruct((A, 1), jnp.float32),
                   jax.ShapeDtypeStruct((1, 128), jnp.int32),
                   jax.ShapeDtypeStruct((T, HI), jnp.int32)),
        in_specs=[pl.BlockSpec((T, H), lambda: (0, 0)),
                  pl.BlockSpec((E, H), lambda: (0, 0))],
        out_specs=(pl.BlockSpec((A, 1), lambda: (0, 0)),
                   pl.BlockSpec((A, 1), lambda: (0, 0)),
                   pl.BlockSpec((1, 128), lambda: (0, 0)),
                   pl.BlockSpec((T, HI), lambda: (0, 0))),
    )(x, gate_w)

    slot1 = slot.reshape(A)
    wf1 = w_flat.reshape(A)

    xs_i, dst, w_slot = pl.kernel(
        _dispatch_kernel,
        out_type=(jax.ShapeDtypeStruct((P, HI), jnp.int32),
                  jax.ShapeDtypeStruct((P,), jnp.int32),
                  jax.ShapeDtypeStruct((P,), jnp.float32)),
        mesh=plsc.VectorSubcoreMesh(core_axis_name="c", subcore_axis_name="s"),
        compiler_params=_SC_PARAMS,
        scratch_types=[pltpu.VMEM((A,), jnp.int32),
                       pltpu.VMEM((A,), jnp.float32),
                       pltpu.VMEM((P,), jnp.int32),
                       pltpu.VMEM((P,), jnp.float32),
                       pltpu.VMEM((2, TPW), jnp.int32),
                       pltpu.VMEM((TPW, HI), jnp.int32),
                       pltpu.SemaphoreType.DMA,
                       pltpu.SemaphoreType.DMA,
                       pltpu.SemaphoreType.DMA],
    )(xbi, slot1, wf1)

    ws3 = w_slot.reshape(GMAX, 1, TM)

    ys = pl.pallas_call(
        _moe_kernel,
        grid_spec=pltpu.PrefetchScalarGridSpec(
            num_scalar_prefetch=1,
            grid=(GMAX,),
            in_specs=[
                pl.BlockSpec((TM, HI),
                             lambda g, m: (jnp.minimum(g, m[0, 127] - 1), 0)),
                pl.BlockSpec((1, 1, TM),
                             lambda g, m: (jnp.minimum(g, m[0, 127] - 1), 0, 0)),
                pl.BlockSpec((1, F, H), lambda g, m: (m[0, g], 0, 0)),
                pl.BlockSpec((1, F, H), lambda g, m: (m[0, g], 0, 0)),
                pl.BlockSpec((1, H, F), lambda g, m: (m[0, g], 0, 0)),
            ],
            out_specs=pl.BlockSpec(
                (TM, HI), lambda g, m: (jnp.minimum(g, m[0, 127] - 1), 0)),
        ),
        out_shape=jax.ShapeDtypeStruct((P, HI), jnp.int32),
        compiler_params=pltpu.CompilerParams(
            dimension_semantics=("arbitrary",)),
    )(meta, xs_i, ws3, Wg, Wu, Wd)

    r = pl.kernel(
        _combine_scatter_kernel,
        out_type=jax.ShapeDtypeStruct((P, HI), jnp.int32),
        mesh=plsc.VectorSubcoreMesh(core_axis_name="c", subcore_axis_name="s"),
        compiler_params=_SC_PARAMS,
        scratch_types=[pltpu.VMEM((P // NW // CC, CC), jnp.int32),
                       pltpu.VMEM((CC, HI), jnp.int32),
                       pltpu.VMEM((CC, HI), jnp.int32),
                       pltpu.SemaphoreType.DMA,
                       pltpu.SemaphoreType.DMA,
                       pltpu.SemaphoreType.DMA],
    )(ys, dst)
    r3 = r.reshape(P // T, T, HI)

    y = pl.pallas_call(
        _add_kernel,
        grid=(T // TM,),
        out_shape=jax.ShapeDtypeStruct((T, H), jnp.float32),
        in_specs=[pl.BlockSpec((1, TM, HI), lambda t: (0, t, 0)),
                  pl.BlockSpec((1, TM, HI), lambda t: (1, t, 0))],
        out_specs=pl.BlockSpec((TM, H), lambda t: (t, 0)),
    )(r3, r3)

    return y.reshape(orig_shape)
